# Initial kernel scaffold; baseline (speedup 1.0000x reference)
import jax
import jax.numpy as jnp
from jax.experimental import pallas as pl


def kernel(char_emb, tok_emb, conv_w_flat, conv_b, wqkv, bqkv, wo, bo, w1, b1, w2, b2, ln1_g, ln1_b, ln2_g, ln2_b, input_code, input_codechar, inputAd):
    raise NotImplementedError("write your pallas kernel here")


# speedup vs baseline: 2.9416x; 2.9416x over previous
import jax
import jax.numpy as jnp
from jax.experimental import pallas as pl
from jax.experimental.pallas import tpu as pltpu

EMBED = 64
NL_LEN = 128
WO_LEN = 8
HEADS = 8
HEAD_DIM = EMBED // HEADS
FF_HIDDEN = 4 * EMBED
NUM_BLOCKS = 6
HN = HEADS * NL_LEN
NEG_INF = -1e9
NB = 8


def _enc_kernel(code_ref, charT_ref, ad_ref,
                mchar_ref, convb_ref, tokt_ref,
                wqkv_ref, bqkv_ref, wo_ref, bo_ref,
                w1_ref, b1_ref, w2_ref, b2_ref,
                ln1g_ref, ln1b_ref, ln2g_ref, ln2b_ref,
                out_ref):
    N, E = NL_LEN, EMBED
    R = NB * N
    f32 = jnp.float32
    bf16 = jnp.bfloat16

    code = code_ref[...]
    lane = jax.lax.broadcasted_iota(jnp.int32, (NB, N, E), 2)

    oh_tok = (code[:, :, None] == lane).astype(bf16).reshape(R, E)
    x = jnp.dot(oh_tok, tokt_ref[...], preferred_element_type=f32)

    charT = charT_ref[...]
    oh_char = jnp.concatenate(
        [(charT[:, w, :, None] == lane).astype(bf16) for w in range(WO_LEN)],
        axis=2).reshape(R, WO_LEN * E)
    char_em = (jnp.dot(oh_char, mchar_ref[...], preferred_element_type=f32)
               + convb_ref[...])

    colpad = jnp.where(code > 0, 0.0, NEG_INF).astype(f32)
    adb = jnp.where(ad_ref[...] > 0, 0.0, NEG_INF) + colpad[:, None, :]
    bias_tree = jnp.broadcast_to(adb[:, :, None, :],
                                 (NB, N, HEADS, N)).reshape(NB, N, HN)
    bias_pad = jnp.broadcast_to(colpad[:, None, None, :],
                                (NB, 1, HEADS, N)).reshape(NB, 1, HN)

    row_h = jax.lax.broadcasted_iota(jnp.int32, (HN, E), 0) // N
    col_h = jax.lax.broadcasted_iota(jnp.int32, (HN, E), 1) // HEAD_DIM
    head_mask = (row_h == col_h).astype(bf16)

    def layernorm(h, g, b):
        mu = jnp.mean(h, axis=-1, keepdims=True)
        hc = h - mu
        var = jnp.mean(hc * hc, axis=-1, keepdims=True)
        return hc * jax.lax.rsqrt(var + 1e-6) * g + b

    for blk in range(NUM_BLOCKS):
        x = x + char_em

        h = layernorm(x, ln1g_ref[blk], ln1b_ref[blk]).astype(bf16)
        qkv = (jnp.dot(h, wqkv_ref[blk], preferred_element_type=f32)
               + bqkv_ref[blk])
        q = qkv[:, :E].astype(bf16).reshape(NB, N, E)
        k = qkv[:, E:2 * E].astype(bf16).reshape(NB, N, E)
        v = qkv[:, 2 * E:3 * E].astype(bf16).reshape(NB, N, E)
        kp = jnp.concatenate([k] * HEADS, axis=1) * head_mask[None]
        vp = jnp.concatenate([v] * HEADS, axis=1) * head_mask[None]

        s = jax.lax.dot_general(q, kp, (((2,), (2,)), ((0,), (0,))),
                                preferred_element_type=f32)
        s = s + (bias_tree if blk < 3 else bias_pad)
        m = jnp.max(s, axis=-1, keepdims=True)
        p = jnp.exp(s - m)
        o = jax.lax.dot_general(p.astype(bf16), vp,
                                (((2,), (1,)), ((0,), (0,))),
                                preferred_element_type=f32)
        denom = jnp.sum(p.reshape(NB, N, HEADS, N), axis=-1)
        rden = 1.0 / jnp.maximum(denom, 1e-30)
        scale = jnp.broadcast_to(rden[:, :, :, None],
                                 (NB, N, HEADS, HEAD_DIM)).reshape(NB, N, E)
        o = (o * scale).reshape(R, E).astype(bf16)
        x = x + jnp.dot(o, wo_ref[blk], preferred_element_type=f32) + bo_ref[blk]

        h2 = layernorm(x, ln2g_ref[blk], ln2b_ref[blk]).astype(bf16)
        ff = jnp.dot(h2, w1_ref[blk], preferred_element_type=f32) + b1_ref[blk]
        ff = 0.5 * ff * (1.0 + jnp.tanh(0.7978845608028654
                                        * (ff + 0.044715 * ff * ff * ff)))
        x = x + (jnp.dot(ff.astype(bf16), w2_ref[blk],
                         preferred_element_type=f32) + b2_ref[blk])

    out_ref[...] = x.reshape(NB, N, E)


def kernel(char_emb, tok_emb, conv_w_flat, conv_b, wqkv, bqkv, wo, bo,
           w1, b1, w2, b2, ln1_g, ln1_b, ln2_g, ln2_b,
           input_code, input_codechar, inputAd):
    B, N = input_code.shape
    E = EMBED
    bf16 = jnp.bfloat16

    conv_w3 = conv_w_flat.reshape(WO_LEN, E, E)
    mchar = jnp.einsum('ve,weo->wvo', char_emb, conv_w3)
    mchar = jnp.pad(mchar, ((0, 0), (0, E - char_emb.shape[0]), (0, 0)))
    mchar = mchar.reshape(WO_LEN * E, E).astype(bf16)
    tokt = jnp.pad(tok_emb, ((0, E - tok_emb.shape[0]), (0, 0))).astype(bf16)
    charT = jnp.transpose(input_codechar, (0, 2, 1))

    full = lambda a: pl.BlockSpec(a.shape, lambda b, _s=a.ndim: (0,) * _s)
    wq_b, wo_b = wqkv.astype(bf16), wo.astype(bf16)
    w1_b, w2_b = w1.astype(bf16), w2.astype(bf16)

    return pl.pallas_call(
        _enc_kernel,
        out_shape=jax.ShapeDtypeStruct((B, N, E), jnp.float32),
        grid=(B // NB,),
        in_specs=[
            pl.BlockSpec((NB, N), lambda b: (b, 0)),
            pl.BlockSpec((NB, WO_LEN, N), lambda b: (b, 0, 0)),
            pl.BlockSpec((NB, N, N), lambda b: (b, 0, 0)),
            full(mchar), full(conv_b), full(tokt),
            full(wq_b), full(bqkv), full(wo_b), full(bo),
            full(w1_b), full(b1), full(w2_b), full(b2),
            full(ln1_g), full(ln1_b), full(ln2_g), full(ln2_b),
        ],
        out_specs=pl.BlockSpec((NB, N, E), lambda b: (b, 0, 0)),
        compiler_params=pltpu.CompilerParams(
            dimension_semantics=("parallel",)),
    )(input_code, charT, inputAd,
      mchar, conv_b, tokt,
      wq_b, bqkv, wo_b, bo, w1_b, b1, w2_b, b2,
      ln1_g, ln1_b, ln2_g, ln2_b)
